# Initial kernel scaffold; baseline (speedup 1.0000x reference)
import jax
import jax.numpy as jnp
from jax.experimental import pallas as pl


def kernel(inputs, Wg, bg, W1, b1, W2, b2):
    raise NotImplementedError("write your pallas kernel here")


# speedup vs baseline: 3.8669x; 3.8669x over previous
import functools

import jax
import jax.numpy as jnp
from jax.experimental import pallas as pl
from jax.experimental.pallas import tpu as pltpu

E = 8
K = 2
B = 2
S = 2048
D = 1024
F = 4096
T = B * S

M = 256
FB = 2048
NF = F // FB
NT = 40
NPAD = NT * M


def _gate_body(x_ref, wg_ref, bg_ref, w_ref, idx_ref):
    x = x_ref[0]
    logits = jax.lax.dot_general(
        x, wg_ref[...], (((1,), (0,)), ((), ())),
        preferred_element_type=jnp.float32) + bg_ref[...]
    iota = jax.lax.broadcasted_iota(jnp.int32, logits.shape, 1)
    m1 = jnp.max(logits, axis=1, keepdims=True)
    a1 = jnp.min(jnp.where(logits == m1, iota, E), axis=1, keepdims=True)
    l2 = jnp.where(iota == a1, -jnp.inf, logits)
    m2 = jnp.max(l2, axis=1, keepdims=True)
    a2 = jnp.min(jnp.where(l2 == m2, iota, E), axis=1, keepdims=True)
    e1 = jnp.exp(m1 - jnp.max(m1, axis=0, keepdims=True))
    e2 = jnp.exp(m2 - jnp.max(m2, axis=0, keepdims=True))
    w_ref[0, :, 0:1] = e1 / jnp.sum(e1, axis=0, keepdims=True)
    w_ref[0, :, 1:2] = e2 / jnp.sum(e2, axis=0, keepdims=True)
    idx_ref[0, :, 0:1] = a1
    idx_ref[0, :, 1:2] = a2


def _route(inputs, Wg, bg):
    return pl.pallas_call(
        _gate_body,
        grid=(B,),
        in_specs=[
            pl.BlockSpec((1, S, D), lambda b: (b, 0, 0)),
            pl.BlockSpec((D, E), lambda b: (0, 0)),
            pl.BlockSpec((1, E), lambda b: (0, 0)),
        ],
        out_specs=[
            pl.BlockSpec((1, S, K), lambda b: (b, 0, 0)),
            pl.BlockSpec((1, S, K), lambda b: (b, 0, 0)),
        ],
        out_shape=[
            jax.ShapeDtypeStruct((B, S, K), jnp.float32),
            jax.ShapeDtypeStruct((B, S, K), jnp.int32),
        ],
    )(inputs, Wg, bg.reshape(1, E))


def _ffn_body(meta_ref, xs_ref, w1_ref, b1_ref, w2_ref, b2_ref, ws_ref, y_ref):
    t = pl.program_id(0)
    f = pl.program_id(1)

    @pl.when(meta_ref[NT + t] == 1)
    def _():
        x = xs_ref[...]
        h = jax.lax.dot_general(
            x, w1_ref[0], (((1,), (0,)), ((), ())),
            preferred_element_type=jnp.float32) + b1_ref[0]
        h = 0.5 * h * (1.0 + jax.lax.erf(h * 0.7071067811865476))
        part = jax.lax.dot_general(
            h, w2_ref[0], (((1,), (0,)), ((), ())),
            preferred_element_type=jnp.float32)

        @pl.when(f == 0)
        def _():
            y_ref[...] = part + b2_ref[0]

        @pl.when(f > 0)
        def _():
            y_ref[...] = y_ref[...] + part

        @pl.when(f == NF - 1)
        def _():
            y_ref[...] = y_ref[...] * ws_ref[...]


def _grouped_ffn(meta, xs, W1, b1, W2, b2, ws):
    grid_spec = pltpu.PrefetchScalarGridSpec(
        num_scalar_prefetch=1,
        grid=(NT, NF),
        in_specs=[
            pl.BlockSpec((M, D), lambda t, f, m: (t, 0)),
            pl.BlockSpec((1, D, FB), lambda t, f, m: (m[t], 0, f)),
            pl.BlockSpec((1, 1, FB), lambda t, f, m: (m[t], 0, f)),
            pl.BlockSpec((1, FB, D), lambda t, f, m: (m[t], f, 0)),
            pl.BlockSpec((1, 1, D), lambda t, f, m: (m[t], 0, 0)),
            pl.BlockSpec((M, 1), lambda t, f, m: (t, 0)),
        ],
        out_specs=pl.BlockSpec((M, D), lambda t, f, m: (t, 0)),
    )
    return pl.pallas_call(
        _ffn_body,
        grid_spec=grid_spec,
        out_shape=jax.ShapeDtypeStruct((NPAD, D), jnp.float32),
        compiler_params=pltpu.CompilerParams(
            dimension_semantics=("arbitrary", "arbitrary")),
    )(meta, xs, W1, b1.reshape(E, 1, F), W2, b2.reshape(E, 1, D), ws)


def kernel(inputs, Wg, bg, W1, b1, W2, b2):
    x_flat = inputs.reshape(T, D)

    w_bsk, idx_bsk = _route(inputs, Wg, bg)
    wf = w_bsk.reshape(T * K)
    e_flat = idx_bsk.reshape(T * K)

    oh = (e_flat[:, None] == jnp.arange(E, dtype=jnp.int32)).astype(jnp.int32)
    ranks = jnp.cumsum(oh, axis=0)
    counts = ranks[-1]
    rank = jnp.take_along_axis(ranks, e_flat[:, None], axis=1)[:, 0] - 1
    tiles_per_e = (counts + M - 1) // M
    seg_starts = jnp.concatenate(
        [jnp.zeros(1, jnp.int32),
         jnp.cumsum(tiles_per_e).astype(jnp.int32)]) * M
    pos = seg_starts[e_flat] + rank
    arange_a = jnp.arange(T * K, dtype=jnp.int32)
    row_tok = jnp.zeros(NPAD, jnp.int32).at[pos].set(arange_a // K)
    ws = jnp.zeros(NPAD, jnp.float32).at[pos].set(wf)
    num_real = jnp.sum(tiles_per_e).astype(jnp.int32)
    tile_ids = jnp.arange(NT, dtype=jnp.int32)
    tile_eid = jnp.minimum(
        jnp.searchsorted(seg_starts[1:], tile_ids * M, side="right"),
        E - 1).astype(jnp.int32)
    tile_valid = (tile_ids < num_real).astype(jnp.int32)
    meta = jnp.concatenate([tile_eid, tile_valid])

    xs = x_flat[row_tok]

    y = _grouped_ffn(meta, xs, W1, b1, W2, b2, ws.reshape(NPAD, 1))

    pos_tk = pos.reshape(T, K)
    out = y[pos_tk[:, 0]] + y[pos_tk[:, 1]]
    return out.reshape(B, S, D)
